# baseline (speedup 1.0000x reference)
import functools

import jax
import jax.numpy as jnp
from jax import lax
from jax.experimental import pallas as pl
from jax.experimental.pallas import tpu as pltpu
from jax.experimental.pallas import tpu_sc as plsc

N_GENE = 10000
N_DRUG = 10000
D = 128

NC = 2
NS = 16
NW = NC * NS
C = 64
ACC_ROWS = 10240
ROWS_PER_SUB = ACC_ROWS // NS
DUMMY_DST = N_GENE


def _fill_const(ref, nrows, width, val):
  v = jnp.full((16,), val, jnp.float32)

  def body(r, _):
    for c16 in range(width // 16):
      ref[r, pl.ds(c16 * 16, 16)] = v
    return 0

  lax.fori_loop(0, nrows, body, 0)


NBUF = 4


def _run_phase(h_hbm, ech_hbm, out_hbm, nchunks_w,
               ibufs, rows, acc_sh, gsems, ssems, gather=True):
  c = lax.axis_index("c")
  s = lax.axis_index("s")
  wid = s * NC + c

  _fill_const(rows[0], C, D, 0.0)
  for j in range(ROWS_PER_SUB // C):
    pltpu.sync_copy(rows[0], acc_sh.at[pl.ds(s * ROWS_PER_SUB + j * C, C)])
  plsc.subcore_barrier()

  if not gather:
    for b in range(NBUF):
      _fill_const(rows[b], C, D, 1.0)
  for b in range(NBUF):
    pltpu.sync_copy(ech_hbm.at[b * NW + wid], ibufs[b])
    if gather:
      pltpu.async_copy(h_hbm.at[ibufs[b].at[0]], rows[b], gsems[b])

  def outer(j, _):
    for b in range(NBUF):
      if gather:
        pltpu.make_async_copy(h_hbm.at[ibufs[b].at[0]], rows[b],
                              gsems[b]).wait()
      pltpu.async_copy(rows[b], acc_sh.at[ibufs[b].at[1]], ssems[b],
                       add=True)
    for b in range(NBUF):
      i2 = j * NBUF + b + NBUF

      @pl.when(i2 < nchunks_w)
      def _():
        pltpu.make_async_copy(rows[b], acc_sh.at[ibufs[b].at[1]],
                              ssems[b]).wait()
        pltpu.sync_copy(ech_hbm.at[i2 * NW + wid], ibufs[b])
        if gather:
          pltpu.async_copy(h_hbm.at[ibufs[b].at[0]], rows[b], gsems[b])

    return 0

  lax.fori_loop(0, nchunks_w // NBUF, outer, 0)
  for b in range(NBUF):
    pltpu.make_async_copy(rows[b], acc_sh.at[ibufs[b].at[1]],
                          ssems[b]).wait()
  plsc.subcore_barrier()

  pltpu.sync_copy(acc_sh.at[pl.ds(s * ROWS_PER_SUB, ROWS_PER_SUB)],
                  out_hbm.at[c, pl.ds(s * ROWS_PER_SUB, ROWS_PER_SUB)])
  plsc.subcore_barrier()


def _make_sc_layer(nchunks_gg, nchunks_dg, nchunks_gd, gather=True):
  mesh = plsc.VectorSubcoreMesh(core_axis_name="c", subcore_axis_name="s",
                                num_cores=NC, num_subcores=NS)
  f32 = jnp.float32
  outs = [jax.ShapeDtypeStruct((NC, ACC_ROWS, D), f32)] * 3
  scratch = (
      [pltpu.VMEM((2, C), jnp.int32) for _ in range(NBUF)]
      + [pltpu.VMEM((C, D), f32) for _ in range(NBUF)]
      + [pltpu.VMEM_SHARED((ACC_ROWS, D), f32)]
      + [pltpu.SemaphoreType.DMA] * (2 * NBUF)
  )

  def body(hg, hd, egg, edg, egd, *rest):
    pgg, pdg, pgd = rest[:3]
    ibufs = rest[3:3 + NBUF]
    rows = rest[3 + NBUF:3 + 2 * NBUF]
    acc_sh = rest[3 + 2 * NBUF]
    gsems = rest[4 + 2 * NBUF:4 + 3 * NBUF]
    ssems = rest[4 + 3 * NBUF:4 + 4 * NBUF]
    _run_phase(hg, egg, pgg, nchunks_gg, ibufs, rows, acc_sh, gsems, ssems,
               gather=gather)
    _run_phase(hd, edg, pdg, nchunks_dg, ibufs, rows, acc_sh, gsems, ssems,
               gather=gather)
    _run_phase(hg, egd, pgd, nchunks_gd, ibufs, rows, acc_sh, gsems, ssems,
               gather=gather)

  return pl.kernel(body, out_type=outs, mesh=mesh, scratch_types=scratch)


BLK = 1000


def _embed_body(gf, df, wg, wd, hg, hd):
  hg[...] = jnp.dot(gf[...], wg[...], preferred_element_type=jnp.float32)
  hd[...] = jnp.dot(df[...], wd[...], preferred_element_type=jnp.float32)


def _embed(gene_feat, drug_feat, W_emb_gene, W_emb_drug):
  dg = gene_feat.shape[1]
  dd = drug_feat.shape[1]
  return pl.pallas_call(
      _embed_body,
      grid=(N_GENE // BLK,),
      in_specs=[
          pl.BlockSpec((BLK, dg), lambda i: (i, 0)),
          pl.BlockSpec((BLK, dd), lambda i: (i, 0)),
          pl.BlockSpec((dg, D), lambda i: (0, 0)),
          pl.BlockSpec((dd, D), lambda i: (0, 0)),
      ],
      out_specs=[pl.BlockSpec((BLK, D), lambda i: (i, 0))] * 2,
      out_shape=[jax.ShapeDtypeStruct((N_GENE, D), jnp.float32),
                 jax.ShapeDtypeStruct((N_DRUG, D), jnp.float32)],
  )(gene_feat, drug_feat, W_emb_gene, W_emb_drug)


def _layer_body(act, pgg, dgg, pdg, ddg, pgd, dgd, hg, hd,
                wgg, wdg, wgd, wl, b, ng, nd):
  inv_gg = 1.0 / jnp.maximum(dgg[0, :, :1] + dgg[1, :, :1], 1.0)
  inv_dg = 1.0 / jnp.maximum(ddg[0, :, :1] + ddg[1, :, :1], 1.0)
  inv_gd = 1.0 / jnp.maximum(dgd[0, :, :1] + dgd[1, :, :1], 1.0)
  agg = (pgg[0] + pgg[1]) * inv_gg
  adg = (pdg[0] + pdg[1]) * inv_dg
  agd = (pgd[0] + pgd[1]) * inv_gd
  f32 = jnp.float32
  ng_v = (jnp.dot(agg, wgg[...], preferred_element_type=f32)
          + jnp.dot(adg, wdg[...], preferred_element_type=f32)
          + jnp.dot(hg[...], wl[...], preferred_element_type=f32) + b[...])
  nd_v = (jnp.dot(agd, wgd[...], preferred_element_type=f32)
          + jnp.dot(hd[...], wl[...], preferred_element_type=f32) + b[...])
  if act:
    ng_v = jnp.maximum(ng_v, 0.0)
    nd_v = jnp.maximum(nd_v, 0.0)
  ng[...] = ng_v
  nd[...] = nd_v


def _layer_tc(act, pgg, dgg, pdg, ddg, pgd, dgd, hg, hd,
              wgg, wdg, wgd, wl, b):
  part = pl.BlockSpec((NC, BLK, D), lambda i: (0, i, 0))
  degs = part
  feat = pl.BlockSpec((BLK, D), lambda i: (i, 0))
  wmat = pl.BlockSpec((D, D), lambda i: (0, 0))
  return pl.pallas_call(
      functools.partial(_layer_body, act),
      grid=(N_GENE // BLK,),
      in_specs=[part, degs, part, degs, part, degs, feat, feat,
                wmat, wmat, wmat, wmat,
                pl.BlockSpec((1, D), lambda i: (0, 0))],
      out_specs=[feat, feat],
      out_shape=[jax.ShapeDtypeStruct((N_GENE, D), jnp.float32),
                 jax.ShapeDtypeStruct((N_DRUG, D), jnp.float32)],
  )(pgg, dgg, pdg, ddg, pgd, dgd, hg, hd, wgg, wdg, wgd, wl,
    b.reshape(1, D))


def _pad_edges(eidx, step=NW * C * NBUF):
  e = eidx.shape[1]
  epad = -(-e // step) * step
  src = jnp.pad(eidx[0].astype(jnp.int32), (0, epad - e))
  dst = jnp.pad(eidx[1].astype(jnp.int32), (0, epad - e),
                constant_values=DUMMY_DST)
  packed = jnp.stack([src.reshape(-1, C), dst.reshape(-1, C)], axis=1)
  return packed, epad


def kernel(gene_feat, drug_feat, eidx_gg, eidx_dg, eidx_gd,
           W_emb_gene, W_emb_drug,
           W0_gg, W0_dg, W0_gd, W0_loop, b0,
           W1_gg, W1_dg, W1_gd, W1_loop, b1,
           W2_gg, W2_dg, W2_gd, W2_loop, b2):
  egg, e_gg = _pad_edges(eidx_gg)
  edg, e_dg = _pad_edges(eidx_dg)
  egd, e_gd = _pad_edges(eidx_gd)
  ncw_gg = e_gg // (NW * C)
  ncw_dg = e_dg // (NW * C)
  ncw_gd = e_gd // (NW * C)

  scn = _make_sc_layer(ncw_gg, ncw_dg, ncw_gd)

  scd = _make_sc_layer(ncw_gg, ncw_dg, ncw_gd, gather=False)
  ones_tbl = jnp.ones((16, D), jnp.float32)
  deg_gg, deg_dg, deg_gd = scd(ones_tbl, ones_tbl, egg, edg, egd)
  h_g, h_d = _embed(gene_feat, drug_feat, W_emb_gene, W_emb_drug)

  pgg, pdg, pgd = scn(h_g, h_d, egg, edg, egd)
  h_g, h_d = _layer_tc(True, pgg, deg_gg, pdg, deg_dg, pgd, deg_gd,
                       h_g, h_d, W0_gg, W0_dg, W0_gd, W0_loop, b0)

  pgg, pdg, pgd = scn(h_g, h_d, egg, edg, egd)
  h_g, h_d = _layer_tc(True, pgg, deg_gg, pdg, deg_dg, pgd, deg_gd,
                       h_g, h_d, W1_gg, W1_dg, W1_gd, W1_loop, b1)

  pgg, pdg, pgd = scn(h_g, h_d, egg, edg, egd)
  h_g, h_d = _layer_tc(False, pgg, deg_gg, pdg, deg_dg, pgd, deg_gd,
                       h_g, h_d, W2_gg, W2_dg, W2_gd, W2_loop, b2)
  return h_g, h_d


# speedup vs baseline: 1.0363x; 1.0363x over previous
import functools

import jax
import jax.numpy as jnp
from jax import lax
from jax.experimental import pallas as pl
from jax.experimental.pallas import tpu as pltpu
from jax.experimental.pallas import tpu_sc as plsc

N_GENE = 10000
N_DRUG = 10000
D = 128

NC = 2
NS = 16
NW = NC * NS
C = 128
ACC_ROWS = 10240
ROWS_PER_SUB = ACC_ROWS // NS
DUMMY_DST = N_GENE


def _fill_const(ref, nrows, width, val):
  v = jnp.full((16,), val, jnp.float32)

  def body(r, _):
    for c16 in range(width // 16):
      ref[r, pl.ds(c16 * 16, 16)] = v
    return 0

  lax.fori_loop(0, nrows, body, 0)


NBUF = 2
NIB = 2 * NBUF


def _run_phase(h_hbm, ech_hbm, out_hbm, nchunks_w,
               ibufs, rows, acc_sh, gsems, ssems, isems, gather=True):
  c = lax.axis_index("c")
  s = lax.axis_index("s")
  wid = s * NC + c

  _fill_const(rows[0], C, D, 0.0)
  for j in range(ROWS_PER_SUB // C):
    pltpu.sync_copy(rows[0], acc_sh.at[pl.ds(s * ROWS_PER_SUB + j * C, C)])
  plsc.subcore_barrier()

  if not gather:
    for b in range(NBUF):
      _fill_const(rows[b], C, D, 1.0)
  for k in range(NIB):
    pltpu.async_copy(ech_hbm.at[k * NW + wid], ibufs[k], isems[k])
  for b in range(NBUF):
    pltpu.make_async_copy(ech_hbm.at[b * NW + wid], ibufs[b],
                          isems[b]).wait()
    if gather:
      pltpu.async_copy(h_hbm.at[ibufs[b].at[0]], rows[b], gsems[b])

  def outer(j, _):
    for jj in range(NIB // NBUF):
      for b in range(NBUF):
        i = j * NIB + jj * NBUF + b
        k = jj * NBUF + b
        if gather:
          pltpu.make_async_copy(h_hbm.at[ibufs[k].at[0]], rows[b],
                                gsems[b]).wait()
        else:
          @pl.when(i >= NBUF)
          def _():
            pltpu.make_async_copy(ech_hbm.at[i * NW + wid], ibufs[k],
                                  isems[k]).wait()
        pltpu.async_copy(rows[b], acc_sh.at[ibufs[k].at[1]], ssems[b],
                         add=True)
      for b in range(NBUF):
        i = j * NIB + jj * NBUF + b
        k = jj * NBUF + b
        i2 = i + NBUF
        k2 = (k + NBUF) % NIB
        i3 = i + NIB

        @pl.when(i2 < nchunks_w)
        def _():
          pltpu.make_async_copy(rows[b], acc_sh.at[ibufs[k].at[1]],
                                ssems[b]).wait()

          @pl.when(i3 < nchunks_w)
          def _():
            pltpu.async_copy(ech_hbm.at[i3 * NW + wid], ibufs[k], isems[k])

          if gather:
            pltpu.make_async_copy(ech_hbm.at[i2 * NW + wid], ibufs[k2],
                                  isems[k2]).wait()
            pltpu.async_copy(h_hbm.at[ibufs[k2].at[0]], rows[b], gsems[b])

    return 0

  lax.fori_loop(0, nchunks_w // NIB, outer, 0)
  for b in range(NBUF):
    pltpu.make_async_copy(rows[b], acc_sh.at[ibufs[NBUF + b].at[1]],
                          ssems[b]).wait()
  plsc.subcore_barrier()

  pltpu.sync_copy(acc_sh.at[pl.ds(s * ROWS_PER_SUB, ROWS_PER_SUB)],
                  out_hbm.at[c, pl.ds(s * ROWS_PER_SUB, ROWS_PER_SUB)])
  plsc.subcore_barrier()


def _make_sc_layer(nchunks_gg, nchunks_dg, nchunks_gd, gather=True):
  mesh = plsc.VectorSubcoreMesh(core_axis_name="c", subcore_axis_name="s",
                                num_cores=NC, num_subcores=NS)
  f32 = jnp.float32
  outs = [jax.ShapeDtypeStruct((NC, ACC_ROWS, D), f32)] * 3
  scratch = (
      [pltpu.VMEM((2, C), jnp.int32) for _ in range(NIB)]
      + [pltpu.VMEM((C, D), f32) for _ in range(NBUF)]
      + [pltpu.VMEM_SHARED((ACC_ROWS, D), f32)]
      + [pltpu.SemaphoreType.DMA] * (2 * NBUF + NIB)
  )

  def body(hg, hd, egg, edg, egd, *rest):
    pgg, pdg, pgd = rest[:3]
    o = 3
    ibufs = rest[o:o + NIB]
    rows = rest[o + NIB:o + NIB + NBUF]
    acc_sh = rest[o + NIB + NBUF]
    sems = rest[o + NIB + NBUF + 1:]
    gsems = sems[:NBUF]
    ssems = sems[NBUF:2 * NBUF]
    isems = sems[2 * NBUF:]
    _run_phase(hg, egg, pgg, nchunks_gg, ibufs, rows, acc_sh, gsems, ssems,
               isems, gather=gather)
    _run_phase(hd, edg, pdg, nchunks_dg, ibufs, rows, acc_sh, gsems, ssems,
               isems, gather=gather)
    _run_phase(hg, egd, pgd, nchunks_gd, ibufs, rows, acc_sh, gsems, ssems,
               isems, gather=gather)

  return pl.kernel(body, out_type=outs, mesh=mesh, scratch_types=scratch)


BLK = 1000


def _embed_body(gf, df, wg, wd, hg, hd):
  hg[...] = jnp.dot(gf[...], wg[...], preferred_element_type=jnp.float32)
  hd[...] = jnp.dot(df[...], wd[...], preferred_element_type=jnp.float32)


def _embed(gene_feat, drug_feat, W_emb_gene, W_emb_drug):
  dg = gene_feat.shape[1]
  dd = drug_feat.shape[1]
  return pl.pallas_call(
      _embed_body,
      grid=(N_GENE // BLK,),
      in_specs=[
          pl.BlockSpec((BLK, dg), lambda i: (i, 0)),
          pl.BlockSpec((BLK, dd), lambda i: (i, 0)),
          pl.BlockSpec((dg, D), lambda i: (0, 0)),
          pl.BlockSpec((dd, D), lambda i: (0, 0)),
      ],
      out_specs=[pl.BlockSpec((BLK, D), lambda i: (i, 0))] * 2,
      out_shape=[jax.ShapeDtypeStruct((N_GENE, D), jnp.float32),
                 jax.ShapeDtypeStruct((N_DRUG, D), jnp.float32)],
  )(gene_feat, drug_feat, W_emb_gene, W_emb_drug)


def _layer_body(act, pgg, dgg, pdg, ddg, pgd, dgd, hg, hd,
                wgg, wdg, wgd, wl, b, ng, nd):
  inv_gg = 1.0 / jnp.maximum(dgg[0, :, :1] + dgg[1, :, :1], 1.0)
  inv_dg = 1.0 / jnp.maximum(ddg[0, :, :1] + ddg[1, :, :1], 1.0)
  inv_gd = 1.0 / jnp.maximum(dgd[0, :, :1] + dgd[1, :, :1], 1.0)
  agg = (pgg[0] + pgg[1]) * inv_gg
  adg = (pdg[0] + pdg[1]) * inv_dg
  agd = (pgd[0] + pgd[1]) * inv_gd
  f32 = jnp.float32
  ng_v = (jnp.dot(agg, wgg[...], preferred_element_type=f32)
          + jnp.dot(adg, wdg[...], preferred_element_type=f32)
          + jnp.dot(hg[...], wl[...], preferred_element_type=f32) + b[...])
  nd_v = (jnp.dot(agd, wgd[...], preferred_element_type=f32)
          + jnp.dot(hd[...], wl[...], preferred_element_type=f32) + b[...])
  if act:
    ng_v = jnp.maximum(ng_v, 0.0)
    nd_v = jnp.maximum(nd_v, 0.0)
  ng[...] = ng_v
  nd[...] = nd_v


def _layer_tc(act, pgg, dgg, pdg, ddg, pgd, dgd, hg, hd,
              wgg, wdg, wgd, wl, b):
  part = pl.BlockSpec((NC, BLK, D), lambda i: (0, i, 0))
  degs = part
  feat = pl.BlockSpec((BLK, D), lambda i: (i, 0))
  wmat = pl.BlockSpec((D, D), lambda i: (0, 0))
  return pl.pallas_call(
      functools.partial(_layer_body, act),
      grid=(N_GENE // BLK,),
      in_specs=[part, degs, part, degs, part, degs, feat, feat,
                wmat, wmat, wmat, wmat,
                pl.BlockSpec((1, D), lambda i: (0, 0))],
      out_specs=[feat, feat],
      out_shape=[jax.ShapeDtypeStruct((N_GENE, D), jnp.float32),
                 jax.ShapeDtypeStruct((N_DRUG, D), jnp.float32)],
  )(pgg, dgg, pdg, ddg, pgd, dgd, hg, hd, wgg, wdg, wgd, wl,
    b.reshape(1, D))


def _pad_edges(eidx, step=NW * C * NIB):
  e = eidx.shape[1]
  epad = -(-e // step) * step
  src = jnp.pad(eidx[0].astype(jnp.int32), (0, epad - e))
  dst = jnp.pad(eidx[1].astype(jnp.int32), (0, epad - e),
                constant_values=DUMMY_DST)
  packed = jnp.stack([src.reshape(-1, C), dst.reshape(-1, C)], axis=1)
  return packed, epad


def kernel(gene_feat, drug_feat, eidx_gg, eidx_dg, eidx_gd,
           W_emb_gene, W_emb_drug,
           W0_gg, W0_dg, W0_gd, W0_loop, b0,
           W1_gg, W1_dg, W1_gd, W1_loop, b1,
           W2_gg, W2_dg, W2_gd, W2_loop, b2):
  egg, e_gg = _pad_edges(eidx_gg)
  edg, e_dg = _pad_edges(eidx_dg)
  egd, e_gd = _pad_edges(eidx_gd)
  ncw_gg = e_gg // (NW * C)
  ncw_dg = e_dg // (NW * C)
  ncw_gd = e_gd // (NW * C)

  scn = _make_sc_layer(ncw_gg, ncw_dg, ncw_gd)

  scd = _make_sc_layer(ncw_gg, ncw_dg, ncw_gd, gather=False)
  ones_tbl = jnp.ones((16, D), jnp.float32)
  deg_gg, deg_dg, deg_gd = scd(ones_tbl, ones_tbl, egg, edg, egd)
  h_g, h_d = _embed(gene_feat, drug_feat, W_emb_gene, W_emb_drug)

  pgg, pdg, pgd = scn(h_g, h_d, egg, edg, egd)
  h_g, h_d = _layer_tc(True, pgg, deg_gg, pdg, deg_dg, pgd, deg_gd,
                       h_g, h_d, W0_gg, W0_dg, W0_gd, W0_loop, b0)

  pgg, pdg, pgd = scn(h_g, h_d, egg, edg, egd)
  h_g, h_d = _layer_tc(True, pgg, deg_gg, pdg, deg_dg, pgd, deg_gd,
                       h_g, h_d, W1_gg, W1_dg, W1_gd, W1_loop, b1)

  pgg, pdg, pgd = scn(h_g, h_d, egg, edg, egd)
  h_g, h_d = _layer_tc(False, pgg, deg_gg, pdg, deg_dg, pgd, deg_gd,
                       h_g, h_d, W2_gg, W2_dg, W2_gd, W2_loop, b2)
  return h_g, h_d
